# Initial kernel scaffold; baseline (speedup 1.0000x reference)
import jax
import jax.numpy as jnp
from jax.experimental import pallas as pl


def kernel(x, codebooks):
    raise NotImplementedError("write your pallas kernel here")


# speedup vs baseline: 1.0814x; 1.0814x over previous
import functools

import jax
import jax.numpy as jnp
from jax.experimental import pallas as pl

L = 4
K = 1024
D = 32
B = 16384
BETA = 0.01

BB = 512


def _rvq_kernel(x_ref, cb_ref, idx_ref, p0_ref, p1_ref, p2_ref, p3_ref,
                quant_ref, loss_ref):
    p_refs = (p0_ref, p1_ref, p2_ref, p3_ref)
    residual = x_ref[...]
    quantized = jnp.zeros_like(residual)
    iota = jax.lax.broadcasted_iota(jnp.int32, (residual.shape[0], K), 1)
    ones_row = jnp.ones((1, D), dtype=jnp.float32)
    for l in range(L):
        cb = cb_ref[l]
        rn = jnp.sum(residual * residual, axis=1, keepdims=True)
        cn = jax.lax.dot_general(
            ones_row, cb * cb, (((1,), (1,)), ((), ())),
            precision=jax.lax.Precision.HIGHEST,
        )
        mm = jax.lax.dot_general(
            residual, cb, (((1,), (1,)), ((), ())))
        d = (rn - 2.0 * mm) + cn
        dmin = jnp.min(d, axis=1, keepdims=True)
        idx = jnp.min(jnp.where(d == dmin, iota, K), axis=1, keepdims=True)
        e = jnp.exp(dmin - d)
        p = e / jnp.sum(e, axis=1, keepdims=True)
        p_refs[l][...] = p
        idx_ref[:, pl.ds(l, 1)] = idx
        onehot = (iota == idx).astype(jnp.float32)
        q = jax.lax.dot_general(
            onehot, cb, (((1,), (0,)), ((), ())),
            precision=jax.lax.Precision.HIGHEST,
        )
        diff = residual - q
        m = jnp.mean(diff * diff, axis=1, keepdims=True)
        loss_ref[:, pl.ds(l, 1)] = m + BETA * m
        quants = residual + (q - residual)
        residual = residual - quants
        quantized = quantized + quants
    quant_ref[...] = quantized


@jax.jit
def kernel(x, codebooks):
    nb = B // BB
    out_shapes = (
        jax.ShapeDtypeStruct((B, L), jnp.int32),
        jax.ShapeDtypeStruct((B, K), jnp.float32),
        jax.ShapeDtypeStruct((B, K), jnp.float32),
        jax.ShapeDtypeStruct((B, K), jnp.float32),
        jax.ShapeDtypeStruct((B, K), jnp.float32),
        jax.ShapeDtypeStruct((B, D), jnp.float32),
        jax.ShapeDtypeStruct((B, L), jnp.float32),
    )
    small = pl.BlockSpec((BB, L), lambda i: (i, 0))
    big = pl.BlockSpec((BB, K), lambda i: (i, 0))
    idx, p0, p1, p2, p3, quantized, losses = pl.pallas_call(
        _rvq_kernel,
        grid=(nb,),
        in_specs=[
            pl.BlockSpec((BB, D), lambda i: (i, 0)),
            pl.BlockSpec((L, K, D), lambda i: (0, 0, 0)),
        ],
        out_specs=(
            small, big, big, big, big,
            pl.BlockSpec((BB, D), lambda i: (i, 0)),
            small,
        ),
        out_shape=out_shapes,
    )(x, codebooks)
    soft_probs = jnp.stack([p0, p1, p2, p3], axis=-1)
    return idx, soft_probs, quantized, losses


# speedup vs baseline: 1.0979x; 1.0153x over previous
import functools

import jax
import jax.numpy as jnp
from jax.experimental import pallas as pl
from jax.experimental.pallas import tpu as pltpu

L = 4
K = 1024
D = 32
B = 16384
BETA = 0.01

BB = 512
INV_D = 1.0 / D


def _rvq_kernel(x_ref, cb_ref, idx_ref, p0_ref, p1_ref, p2_ref, p3_ref,
                quant_ref, loss_ref, cn_ref):
    p_refs = (p0_ref, p1_ref, p2_ref, p3_ref)

    @pl.when(pl.program_id(0) == 0)
    def _():
        ones_row = jnp.ones((1, D), dtype=jnp.float32)
        for l in range(L):
            cb = cb_ref[l]
            cn_ref[l] = jax.lax.dot_general(
                ones_row, cb * cb, (((1,), (1,)), ((), ())),
                precision=jax.lax.Precision.HIGHEST,
            )

    residual = x_ref[...]
    quantized = jnp.zeros_like(residual)
    iota = jax.lax.broadcasted_iota(jnp.int32, (residual.shape[0], K), 1)
    for l in range(L):
        cb = cb_ref[l]
        rn = jnp.sum(residual * residual, axis=1, keepdims=True)
        mm = jax.lax.dot_general(
            residual, cb, (((1,), (1,)), ((), ())))
        d = (rn - 2.0 * mm) + cn_ref[l]
        dmin = jnp.min(d, axis=1, keepdims=True)
        idx = jnp.min(jnp.where(d == dmin, iota, K), axis=1, keepdims=True)
        e = jnp.exp(dmin - d)
        p = e / jnp.sum(e, axis=1, keepdims=True)
        p_refs[l][...] = p
        idx_ref[:, pl.ds(l, 1)] = idx
        m = dmin * INV_D
        loss_ref[:, pl.ds(l, 1)] = m + BETA * m
        onehot = (iota == idx).astype(jnp.float32)
        q = jax.lax.dot_general(
            onehot, cb, (((1,), (0,)), ((), ())),
            precision=jax.lax.Precision.HIGHEST,
        )
        quants = residual + (q - residual)
        residual = residual - quants
        quantized = quantized + quants
    quant_ref[...] = quantized


@jax.jit
def kernel(x, codebooks):
    nb = B // BB
    out_shapes = (
        jax.ShapeDtypeStruct((B, L), jnp.int32),
        jax.ShapeDtypeStruct((B, K), jnp.float32),
        jax.ShapeDtypeStruct((B, K), jnp.float32),
        jax.ShapeDtypeStruct((B, K), jnp.float32),
        jax.ShapeDtypeStruct((B, K), jnp.float32),
        jax.ShapeDtypeStruct((B, D), jnp.float32),
        jax.ShapeDtypeStruct((B, L), jnp.float32),
    )
    small = pl.BlockSpec((BB, L), lambda i: (i, 0))
    big = pl.BlockSpec((BB, K), lambda i: (i, 0))
    idx, p0, p1, p2, p3, quantized, losses = pl.pallas_call(
        _rvq_kernel,
        grid=(nb,),
        in_specs=[
            pl.BlockSpec((BB, D), lambda i: (i, 0)),
            pl.BlockSpec((L, K, D), lambda i: (0, 0, 0)),
        ],
        out_specs=(
            small, big, big, big, big,
            pl.BlockSpec((BB, D), lambda i: (i, 0)),
            small,
        ),
        out_shape=out_shapes,
        scratch_shapes=[pltpu.VMEM((L, 1, K), jnp.float32)],
    )(x, codebooks)
    soft_probs = jnp.stack([p0, p1, p2, p3], axis=-1)
    return idx, soft_probs, quantized, losses
